# baseline (speedup 1.0000x reference)
import functools

import jax
import jax.numpy as jnp
from jax import lax
from jax.experimental import pallas as pl
from jax.experimental.pallas import tpu as pltpu
from jax.experimental.pallas import tpu_sc as plsc

NUM_EMBEDDINGS = 8192
EMBEDDING_DIM = 32


_BT = 896
_BK = 512


def _vq_argmin_body(z_ref, cb_ref, idx_ref, best_ref, bidx_ref):
    j = pl.program_id(1)
    nj = pl.num_programs(1)
    z = z_ref[...]
    c = cb_ref[...]
    cn = jnp.sum(c * c, axis=1)
    zn = jnp.sum(z * z, axis=1)
    zc = lax.dot_general(
        c.astype(jnp.bfloat16), z.astype(jnp.bfloat16),
        (((1,), (1,)), ((), ())), preferred_element_type=jnp.float32)
    st = cn[:, None] - 2.0 * zc
    lmin = jnp.min(st, axis=0)
    row = lax.broadcasted_iota(jnp.int32, st.shape, 0)
    cand = jnp.where(st == lmin[None, :], row, jnp.int32(0x7FFFFFFF))
    larg = jnp.min(cand, axis=0) + j * _BK

    @pl.when(j == 0)
    def _():
        best_ref[...] = lmin
        bidx_ref[...] = larg

    @pl.when(j > 0)
    def _():
        better = lmin < best_ref[...]
        best_ref[...] = jnp.where(better, lmin, best_ref[...])
        bidx_ref[...] = jnp.where(better, larg, bidx_ref[...])

    @pl.when(j == nj - 1)
    def _():
        idx_ref[0, 0, :] = bidx_ref[...]


def _vq_argmin(z, codebook):
    n, d = z.shape
    k = codebook.shape[0]
    ni, nj = n // _BT, k // _BK
    idx3 = pl.pallas_call(
        _vq_argmin_body,
        grid=(ni, nj),
        in_specs=[
            pl.BlockSpec((_BT, d), lambda i, j: (i, 0)),
            pl.BlockSpec((_BK, d), lambda i, j: (j, 0)),
        ],
        out_specs=pl.BlockSpec((1, 1, _BT), lambda i, j: (i, 0, 0)),
        out_shape=jax.ShapeDtypeStruct((ni, 1, _BT), jnp.int32),
        scratch_shapes=[
            pltpu.VMEM((_BT,), jnp.float32),
            pltpu.VMEM((_BT,), jnp.int32),
        ],
    )(z, codebook)
    return idx3.reshape(n)


def _sc_gather(table, idx):
    n = idx.shape[0]
    d_orig = table.shape[1]
    table = jnp.pad(table, ((0, 0), (0, 128 - d_orig)))
    d = 128
    info = plsc.get_sparse_core_info()
    nw = info.num_cores * info.num_subcores
    b_per_w = n // nw
    mesh = plsc.VectorSubcoreMesh(core_axis_name="c", subcore_axis_name="s")

    @functools.partial(
        pl.kernel, mesh=mesh,
        out_type=jax.ShapeDtypeStruct((n, d), jnp.float32),
        scratch_types=[
            pltpu.VMEM((b_per_w,), jnp.int32),
            pltpu.VMEM((b_per_w, d), jnp.float32),
            pltpu.SemaphoreType.DMA,
        ],
    )
    def gather_k(table_hbm, idx_hbm, out_hbm, idx_v, rows_v, sem):
        wid = lax.axis_index("s") * info.num_cores + lax.axis_index("c")
        base = wid * b_per_w
        pltpu.sync_copy(idx_hbm.at[pl.ds(base, b_per_w)], idx_v)
        pltpu.async_copy(table_hbm.at[idx_v], rows_v, sem).wait()
        pltpu.sync_copy(rows_v, out_hbm.at[pl.ds(base, b_per_w)])

    return gather_k(table, idx)[:, :d_orig]


_BM = 1792


def _mm_body(w_ref, b_ref, p_ref, o_ref, *, relu):
    acc = lax.dot_general(
        w_ref[0], p_ref[0], (((1,), (0,)), ((), ())),
        preferred_element_type=jnp.float32)
    acc = acc + b_ref[0][:, None]
    o_ref[0] = jnp.maximum(acc, 0.0) if relu else acc


def _mm(w3, bias, p3, relu):
    g_, o_, k_ = w3.shape
    m_ = p3.shape[2]
    nm = m_ // _BM
    return pl.pallas_call(
        functools.partial(_mm_body, relu=relu),
        grid=(g_, nm),
        in_specs=[
            pl.BlockSpec((1, o_, k_), lambda g, m: (g, 0, 0)),
            pl.BlockSpec((1, o_), lambda g, m: (0, 0)),
            pl.BlockSpec((1, k_, _BM), lambda g, m: (g, 0, m)),
        ],
        out_specs=pl.BlockSpec((1, o_, _BM), lambda g, m: (g, 0, m)),
        out_shape=jax.ShapeDtypeStruct((g_, o_, m_), jnp.float32),
    )(w3, bias.reshape(1, o_), p3)


def _conv_patches(x, kh, kw, stride, pad):
    B, C, H, W = x.shape
    Ho = (H + 2 * pad - kh) // stride + 1
    Wo = (W + 2 * pad - kw) // stride + 1
    xp = jnp.pad(x, ((0, 0), (0, 0), (pad, pad), (pad, pad)))
    xt = jnp.transpose(xp, (1, 0, 2, 3))
    taps = [xt[:, :, ty:ty + stride * Ho:stride, tx:tx + stride * Wo:stride]
            for ty in range(kh) for tx in range(kw)]
    P = jnp.stack(taps, axis=1)
    return P.reshape(1, C * kh * kw, B * Ho * Wo), (B, Ho, Wo)


def _conv(x, w, b, stride, pad, relu):
    O, I, kh, kw = w.shape
    P, (B, Ho, Wo) = _conv_patches(x, kh, kw, stride, pad)
    out = _mm(w.reshape(1, O, I * kh * kw), b, P, relu)
    return jnp.transpose(out.reshape(O, B, Ho, Wo), (1, 0, 2, 3))


def _deconv(x, w, b, relu):
    B, C, H, W = x.shape
    O = w.shape[0]
    xp = jnp.pad(x, ((0, 0), (0, 0), (1, 1), (1, 1)))
    xt = jnp.transpose(xp, (1, 0, 2, 3))
    ps, ws = [], []
    for py in range(2):
        for px in range(2):
            taps = [xt[:, :, py + a:py + a + H, px + bb:px + bb + W]
                    for a in range(2) for bb in range(2)]
            ps.append(jnp.stack(taps, axis=1).reshape(C * 4, B * H * W))
            ws.append(w[:, :, py::2, px::2].reshape(O, C * 4))
    P = jnp.stack(ps, axis=0)
    W3 = jnp.stack(ws, axis=0)
    out = _mm(W3, b, P, relu)
    st = out.reshape(2, 2, O, B, H, W)
    return jnp.transpose(st, (3, 2, 4, 0, 5, 1)).reshape(B, O, 2 * H, 2 * W)


def kernel(x, ew1, eb1, ew2, eb2, ew3, eb3, codebook,
           dw1, db1, dw2, db2, dw3, db3):
    h = _conv(x, ew1, eb1, 2, 1, relu=True)
    h = _conv(h, ew2, eb2, 2, 1, relu=True)
    latent = _conv(h, ew3, eb3, 1, 1, relu=False)
    B, D, H, W = latent.shape

    z = jnp.transpose(latent, (0, 2, 3, 1)).reshape(-1, D)
    indices = _vq_argmin(z, codebook)
    q_flat = _sc_gather(codebook, indices)
    quantized = jnp.transpose(q_flat.reshape(B, H, W, D), (0, 3, 1, 2))
    embedding_indices = indices.reshape(B, H, W)

    dec_in = latent + (quantized - latent)
    d = _conv(dec_in, dw1, db1, 1, 1, relu=True)
    d = _deconv(d, dw2, db2, relu=True)
    x_hat = _deconv(d, dw3, db3, relu=False)
    return (x_hat, quantized, latent, embedding_indices)
